# baseline (speedup 1.0000x reference)
import functools

import jax
import jax.numpy as jnp
from jax import lax
from jax.experimental import pallas as pl
from jax.experimental.pallas import tpu as pltpu
from jax.experimental.pallas import tpu_sc as plsc

_N = 4194304
_PHASE_BINS = 64
_MAG_BINS = 1024
_TWO_PI_OVER_P = 2.0 * 3.141592653589793 / _PHASE_BINS
_INV_MM1 = 1.0 / (_MAG_BINS - 1)

_NC = 2
_NS = 16
_NW = _NC * _NS
_LANES = 16
_PER_W = _N // _NW
_CHUNK = 4096
_NCHUNK = _PER_W // _CHUNK
_NPAIRS = _NCHUNK // 2


def _sc_body(cp_hbm, cm_hbm, sp_hbm, sm_hbm, cos_hbm, sin_hbm, mag_hbm,
             phase_hbm, mago_hbm, sig_hbm, part_hbm, gp_hbm, gm_hbm,
             *scratch):
    ins = (scratch[0:4], scratch[4:8])
    outs = (scratch[8:13], scratch[13:18])
    cosv, sinv, magv, accv = scratch[18:22]
    in_sems = scratch[22:24]
    out_sems = scratch[24:26]
    in_hbm = (cp_hbm, cm_hbm, sp_hbm, sm_hbm)
    out_hbm = (phase_hbm, mago_hbm, sig_hbm, gp_hbm, gm_hbm)

    wid = lax.axis_index("c") * _NS + lax.axis_index("s")
    base = wid * _PER_W

    pltpu.sync_copy(cos_hbm, cosv)
    pltpu.sync_copy(sin_hbm, sinv)
    pltpu.sync_copy(mag_hbm, magv)

    def start_in(g, b):
        sl = pl.ds(base + g * _CHUNK, _CHUNK)
        for hbm, buf in zip(in_hbm, ins[b]):
            pltpu.async_copy(hbm.at[sl], buf, in_sems[b])

    def wait_in(b):
        for hbm, buf in zip(in_hbm, ins[b]):
            pltpu.make_async_copy(hbm.at[pl.ds(0, _CHUNK)], buf,
                                  in_sems[b]).wait()

    def start_out(g, b):
        sl = pl.ds(base + g * _CHUNK, _CHUNK)
        for hbm, buf in zip(out_hbm, outs[b]):
            pltpu.async_copy(buf, hbm.at[sl], out_sems[b])

    def wait_out(b):
        for hbm, buf in zip(out_hbm, outs[b]):
            pltpu.make_async_copy(buf, hbm.at[pl.ds(0, _CHUNK)],
                                  out_sems[b]).wait()

    def compute(b, acc):
        cpv, cmv, spv, smv = ins[b]
        pov, mov, sigv, gpv, gmv = outs[b]

        def vec_body(i, acc_in):
            vs = pl.ds(i * _LANES, _LANES)
            p = (cpv[vs] + spv[vs]) & (_PHASE_BINS - 1)
            mg = (cmv[vs] + smv[vs]) & (_MAG_BINS - 1)
            pov[vs] = p
            mov[vs] = mg
            c = plsc.load_gather(cosv, [p])
            s = plsc.load_gather(sinv, [p])
            m = plsc.load_gather(magv, [mg])
            sig = c * m
            sigv[vs] = sig
            gpv[vs] = (s * m) * (-_TWO_PI_OVER_P)
            gmv[vs] = sig * _INV_MM1
            return acc_in + sig

        return lax.fori_loop(0, _CHUNK // _LANES, vec_body, acc, unroll=8)

    start_in(0, 0)

    def pair_body(tt, acc):
        g0 = 2 * tt
        start_in(g0 + 1, 1)
        wait_in(0)

        @pl.when(tt > 0)
        def _():
            wait_out(0)

        acc0 = compute(0, acc)
        start_out(g0, 0)

        @pl.when(tt < _NPAIRS - 1)
        def _():
            start_in(g0 + 2, 0)

        wait_in(1)

        @pl.when(tt > 0)
        def _():
            wait_out(1)

        acc1 = compute(1, acc0)
        start_out(g0 + 1, 1)
        return acc1

    acc = lax.fori_loop(0, _NPAIRS, pair_body,
                        jnp.zeros((_LANES,), jnp.float32))
    wait_out(0)
    wait_out(1)
    accv[...] = acc
    pltpu.sync_copy(accv, part_hbm.at[wid])


_sc_call = functools.partial(
    pl.kernel,
    out_type=(
        jax.ShapeDtypeStruct((_N,), jnp.int32),
        jax.ShapeDtypeStruct((_N,), jnp.int32),
        jax.ShapeDtypeStruct((_N,), jnp.float32),
        jax.ShapeDtypeStruct((_NW, _LANES), jnp.float32),
        jax.ShapeDtypeStruct((_N,), jnp.float32),
        jax.ShapeDtypeStruct((_N,), jnp.float32),
    ),
    mesh=plsc.VectorSubcoreMesh(core_axis_name="c", subcore_axis_name="s"),
    compiler_params=pltpu.CompilerParams(needs_layout_passes=False),
    scratch_types=(
        [pltpu.VMEM((_CHUNK,), jnp.int32)] * 8
        + [pltpu.VMEM((_CHUNK,), jnp.int32)] * 2
        + [pltpu.VMEM((_CHUNK,), jnp.float32)] * 3
        + [pltpu.VMEM((_CHUNK,), jnp.int32)] * 2
        + [pltpu.VMEM((_CHUNK,), jnp.float32)] * 3
        + [pltpu.VMEM((_PHASE_BINS,), jnp.float32),
           pltpu.VMEM((_PHASE_BINS,), jnp.float32),
           pltpu.VMEM((_MAG_BINS,), jnp.float32),
           pltpu.VMEM((_LANES,), jnp.float32)]
        + [pltpu.SemaphoreType.DMA] * 4
    ),
)(_sc_body)


def kernel(ctx_phase_idx, ctx_mag_idx, self_phase_idx, self_mag_idx,
           cos_table, sin_table, mag_table):
    phase_out, mag_out, signal, parts, grad_phase, grad_mag = _sc_call(
        ctx_phase_idx, ctx_mag_idx, self_phase_idx, self_mag_idx,
        cos_table, sin_table, mag_table)
    strength = jnp.sum(parts)
    return (phase_out, mag_out, signal, strength, grad_phase, grad_mag)


# speedup vs baseline: 1.9774x; 1.9774x over previous
import functools

import jax
import jax.numpy as jnp
from jax import lax
from jax.experimental import pallas as pl
from jax.experimental.pallas import tpu as pltpu
from jax.experimental.pallas import tpu_sc as plsc

_N = 4194304
_PHASE_BINS = 64
_MAG_BINS = 1024
_TWO_PI_OVER_P = 2.0 * 3.141592653589793 / _PHASE_BINS
_INV_MM1 = 1.0 / (_MAG_BINS - 1)

_NC = 2
_NS = 16
_NW = _NC * _NS
_LANES = 16
_PER_W = _N // _NW
_CHUNK = 4096
_NCHUNK = _PER_W // _CHUNK
_NPAIRS = _NCHUNK // 2


def _sc_body(cp_hbm, cm_hbm, sp_hbm, sm_hbm, cos_hbm, sin_hbm, mag_hbm,
             phase_hbm, mago_hbm, sig_hbm, part_hbm, gp_hbm, gm_hbm,
             *scratch):
    ins = (scratch[0:4], scratch[4:8])
    outs = (scratch[8:13], scratch[13:18])
    cosv, sinv, magv, accv = scratch[18:22]
    in_sems = scratch[22:24]
    out_sems = scratch[24:26]
    in_hbm = (cp_hbm, cm_hbm, sp_hbm, sm_hbm)
    out_hbm = (phase_hbm, mago_hbm, sig_hbm, gp_hbm, gm_hbm)

    wid = lax.axis_index("c") * _NS + lax.axis_index("s")
    base = wid * _PER_W

    pltpu.sync_copy(cos_hbm, cosv)
    pltpu.sync_copy(sin_hbm, sinv)
    pltpu.sync_copy(mag_hbm, magv)

    def start_in(g, b):
        sl = pl.ds(base + g * _CHUNK, _CHUNK)
        for hbm, buf in zip(in_hbm, ins[b]):
            pltpu.async_copy(hbm.at[sl], buf, in_sems[b])

    def wait_in(b):
        for hbm, buf in zip(in_hbm, ins[b]):
            pltpu.make_async_copy(hbm.at[pl.ds(0, _CHUNK)], buf,
                                  in_sems[b]).wait()

    def start_out(g, b):
        sl = pl.ds(base + g * _CHUNK, _CHUNK)
        for hbm, buf in zip(out_hbm, outs[b]):
            pltpu.async_copy(buf, hbm.at[sl], out_sems[b])

    def wait_out(b):
        for hbm, buf in zip(out_hbm, outs[b]):
            pltpu.make_async_copy(buf, hbm.at[pl.ds(0, _CHUNK)],
                                  out_sems[b]).wait()

    def compute(b, acc):
        cpv, cmv, spv, smv = ins[b]
        pov, mov, sigv, gpv, gmv = outs[b]

        @plsc.parallel_loop(0, _CHUNK, step=_LANES, unroll=4, carry=acc)
        def vec_loop(i, acc_in):
            vs = pl.ds(i, _LANES)
            p = (cpv[vs] + spv[vs]) & (_PHASE_BINS - 1)
            mg = (cmv[vs] + smv[vs]) & (_MAG_BINS - 1)
            pov[vs] = p
            mov[vs] = mg
            c = plsc.load_gather(cosv, [p])
            s = plsc.load_gather(sinv, [p])
            m = plsc.load_gather(magv, [mg])
            sig = c * m
            sigv[vs] = sig
            gpv[vs] = (s * m) * (-_TWO_PI_OVER_P)
            gmv[vs] = sig * _INV_MM1
            return acc_in + sig

        return vec_loop

    start_in(0, 0)

    def pair_body(tt, acc):
        g0 = 2 * tt
        start_in(g0 + 1, 1)
        wait_in(0)

        @pl.when(tt > 0)
        def _():
            wait_out(0)

        acc0 = compute(0, acc)
        start_out(g0, 0)

        @pl.when(tt < _NPAIRS - 1)
        def _():
            start_in(g0 + 2, 0)

        wait_in(1)

        @pl.when(tt > 0)
        def _():
            wait_out(1)

        acc1 = compute(1, acc0)
        start_out(g0 + 1, 1)
        return acc1

    acc = lax.fori_loop(0, _NPAIRS, pair_body,
                        jnp.zeros((_LANES,), jnp.float32))
    wait_out(0)
    wait_out(1)
    accv[...] = acc
    pltpu.sync_copy(accv, part_hbm.at[wid])


_sc_call = functools.partial(
    pl.kernel,
    out_type=(
        jax.ShapeDtypeStruct((_N,), jnp.int32),
        jax.ShapeDtypeStruct((_N,), jnp.int32),
        jax.ShapeDtypeStruct((_N,), jnp.float32),
        jax.ShapeDtypeStruct((_NW, _LANES), jnp.float32),
        jax.ShapeDtypeStruct((_N,), jnp.float32),
        jax.ShapeDtypeStruct((_N,), jnp.float32),
    ),
    mesh=plsc.VectorSubcoreMesh(core_axis_name="c", subcore_axis_name="s"),
    compiler_params=pltpu.CompilerParams(needs_layout_passes=False),
    scratch_types=(
        [pltpu.VMEM((_CHUNK,), jnp.int32)] * 8
        + [pltpu.VMEM((_CHUNK,), jnp.int32)] * 2
        + [pltpu.VMEM((_CHUNK,), jnp.float32)] * 3
        + [pltpu.VMEM((_CHUNK,), jnp.int32)] * 2
        + [pltpu.VMEM((_CHUNK,), jnp.float32)] * 3
        + [pltpu.VMEM((_PHASE_BINS,), jnp.float32),
           pltpu.VMEM((_PHASE_BINS,), jnp.float32),
           pltpu.VMEM((_MAG_BINS,), jnp.float32),
           pltpu.VMEM((_LANES,), jnp.float32)]
        + [pltpu.SemaphoreType.DMA] * 4
    ),
)(_sc_body)


def kernel(ctx_phase_idx, ctx_mag_idx, self_phase_idx, self_mag_idx,
           cos_table, sin_table, mag_table):
    phase_out, mag_out, signal, parts, grad_phase, grad_mag = _sc_call(
        ctx_phase_idx, ctx_mag_idx, self_phase_idx, self_mag_idx,
        cos_table, sin_table, mag_table)
    strength = jnp.sum(parts)
    return (phase_out, mag_out, signal, strength, grad_phase, grad_mag)
